# baseline (speedup 1.0000x reference)
import functools

import jax
import jax.numpy as jnp
from jax import lax
from jax.experimental import pallas as pl
from jax.experimental.pallas import tpu as pltpu
from jax.experimental.pallas import tpu_sc as plsc

BN = 256


NCHUNK = 4


def _tie_threshold(m):
    mc = jnp.maximum(m, 0.0)
    sm = jnp.sqrt(mc)
    t = mc
    t0_bits = lax.bitcast_convert_type(sm * sm, jnp.int32)
    for db in range(-2, 4):
        cand = lax.bitcast_convert_type(t0_bits + db, jnp.float32)
        ok = (jnp.sqrt(cand) == sm) & (cand > t)
        t = jnp.where(ok, cand, t)
    return sm, t


def _argmin_body(x2_ref, cb_ref, a2_ref, b2_ref, idx_ref):
    x2 = x2_ref[...]
    k = cb_ref.shape[0]
    kc = k // NCHUNK
    a2 = a2_ref[...]
    keys, firsts = [], []
    for c in range(NCHUNK):
        cb_c = cb_ref[c * kc:(c + 1) * kc, :]
        b2_c = b2_ref[:, c * kc:(c + 1) * kc]
        s2 = lax.dot_general(
            x2, cb_c, (((1,), (1,)), ((), ())),
            preferred_element_type=jnp.float32)
        d2 = (a2 + b2_c) - s2
        m = jnp.min(d2, axis=1, keepdims=True)
        sm, t = _tie_threshold(m)
        iota = (lax.broadcasted_iota(jnp.int32, (1, kc), 1)
                + c * kc).astype(jnp.float32)
        f = jnp.min(jnp.where(d2 <= t, iota, float(k)),
                    axis=1, keepdims=True)
        keys.append(sm)
        firsts.append(f)
    s_all = jnp.concatenate(keys, axis=1)
    f_all = jnp.concatenate(firsts, axis=1)
    s_min = jnp.min(s_all, axis=1, keepdims=True)
    idx = jnp.min(jnp.where(s_all == s_min, f_all, float(k)), axis=1)
    idx_ref[...] = idx.astype(jnp.int32)


def _tc_argmin(flat, codebook, a2, b2):
    n, d = flat.shape
    k = codebook.shape[0]
    grid = (n // BN,)
    return pl.pallas_call(
        _argmin_body,
        grid=grid,
        in_specs=[
            pl.BlockSpec((BN, d), lambda i: (i, 0)),
            pl.BlockSpec((k, d), lambda i: (0, 0)),
            pl.BlockSpec((BN, 1), lambda i: (i, 0)),
            pl.BlockSpec((1, k), lambda i: (0, 0)),
        ],
        out_specs=pl.BlockSpec((BN,), lambda i: (i,)),
        out_shape=jax.ShapeDtypeStruct((n,), jnp.int32),
    )(flat, codebook, a2, b2)


_NC, _NS = 2, 16
_NW = _NC * _NS
_CHUNK = 96


def _make_sc_gather(n, d):
    per_w = n // _NW
    n_chunks = per_w // _CHUNK
    mesh = plsc.VectorSubcoreMesh(core_axis_name="c", subcore_axis_name="s")

    @functools.partial(
        pl.kernel, mesh=mesh,
        out_type=jax.ShapeDtypeStruct((n, d), jnp.float32),
        scratch_types=[
            pltpu.VMEM((_CHUNK,), jnp.int32),
            pltpu.VMEM((_CHUNK, d), jnp.float32),
            pltpu.SemaphoreType.DMA,
        ],
    )
    def gather(table_hbm, idx_hbm, out_hbm, idx_v, rows_v, sem):
        wid = lax.axis_index("s") * _NC + lax.axis_index("c")
        base = wid * per_w
        for c in range(n_chunks):
            off = base + c * _CHUNK
            pltpu.sync_copy(idx_hbm.at[pl.ds(off, _CHUNK)], idx_v)
            pltpu.async_copy(table_hbm.at[idx_v], rows_v, sem).wait()
            pltpu.sync_copy(rows_v, out_hbm.at[pl.ds(off, _CHUNK)])

    return gather


def kernel(input, codebook):
    batch_shape = input.shape[:-1]
    d = input.shape[-1]
    flat = input.reshape(-1, d)
    a2 = jnp.sum(flat * flat, axis=-1, keepdims=True)
    b2 = jnp.sum(codebook * codebook, axis=-1)[None, :]
    idx_flat = _tc_argmin(2.0 * flat, codebook, a2, b2)
    embed = _make_sc_gather(flat.shape[0], d)(
        codebook, idx_flat)
    return embed.reshape(*batch_shape, d), idx_flat.reshape(batch_shape)


# speedup vs baseline: 1.2785x; 1.2785x over previous
import functools

import jax
import jax.numpy as jnp
from jax import lax
from jax.experimental import pallas as pl
from jax.experimental.pallas import tpu as pltpu
from jax.experimental.pallas import tpu_sc as plsc

BN = 256


def _argmin_body(x_ref, cb_ref, a2_ref, b2_ref, idx_ref):
    x = x_ref[...]
    x2 = x + x
    cb = cb_ref[...]
    s2 = lax.dot_general(
        x2, cb, (((1,), (1,)), ((), ())),
        preferred_element_type=jnp.float32)
    d2 = (a2_ref[...] + b2_ref[...]) - s2
    k = d2.shape[1]
    m = jnp.min(d2, axis=1, keepdims=True)
    mc = jnp.maximum(m, 0.0)
    sm = jnp.sqrt(mc)
    base_bits = lax.bitcast_convert_type(sm * sm, jnp.int32) - 2
    count = jnp.zeros_like(base_bits)
    for db in range(6):
        cand = lax.bitcast_convert_type(base_bits + db, jnp.float32)
        count = count + (jnp.sqrt(cand) <= sm).astype(jnp.int32)
    t_bits = base_bits + count - 1
    t = lax.bitcast_convert_type(t_bits, jnp.float32)
    t = jnp.where(mc > 0.0, jnp.maximum(t, mc), 0.0)
    iota = lax.broadcasted_iota(jnp.int32, (1, k), 1).astype(jnp.float32)
    idx = jnp.min(jnp.where(d2 <= t, iota, float(k)), axis=1)
    idx_ref[...] = idx.astype(jnp.int32)


def _tc_argmin(flat, codebook, a2, b2):
    n, d = flat.shape
    k = codebook.shape[0]
    grid = (n // BN,)
    return pl.pallas_call(
        _argmin_body,
        grid=grid,
        in_specs=[
            pl.BlockSpec((BN, d), lambda i: (i, 0)),
            pl.BlockSpec((k, d), lambda i: (0, 0)),
            pl.BlockSpec((BN, 1), lambda i: (i, 0)),
            pl.BlockSpec((1, k), lambda i: (0, 0)),
        ],
        out_specs=pl.BlockSpec((BN,), lambda i: (i,)),
        out_shape=jax.ShapeDtypeStruct((n,), jnp.int32),
    )(flat, codebook, a2, b2)


_NC, _NS = 2, 16
_NW = _NC * _NS
_CHUNK = 96


def _make_sc_gather(n, d):
    per_w = n // _NW
    n_chunks = per_w // _CHUNK
    mesh = plsc.VectorSubcoreMesh(core_axis_name="c", subcore_axis_name="s")

    @functools.partial(
        pl.kernel, mesh=mesh,
        out_type=jax.ShapeDtypeStruct((n, d), jnp.float32),
        scratch_types=[
            pltpu.VMEM((_CHUNK,), jnp.int32),
            pltpu.VMEM((_CHUNK, d), jnp.float32),
            pltpu.SemaphoreType.DMA,
        ],
    )
    def gather(table_hbm, idx_hbm, out_hbm, idx_v, rows_v, sem):
        wid = lax.axis_index("s") * _NC + lax.axis_index("c")
        base = wid * per_w
        for c in range(n_chunks):
            off = base + c * _CHUNK
            pltpu.sync_copy(idx_hbm.at[pl.ds(off, _CHUNK)], idx_v)
            pltpu.async_copy(table_hbm.at[idx_v], rows_v, sem).wait()
            pltpu.sync_copy(rows_v, out_hbm.at[pl.ds(off, _CHUNK)])

    return gather


def kernel(input, codebook):
    batch_shape = input.shape[:-1]
    d = input.shape[-1]
    flat = input.reshape(-1, d)
    a2 = jnp.sum(flat * flat, axis=-1, keepdims=True)
    b2 = jnp.sum(codebook * codebook, axis=-1)[None, :]
    idx_flat = _tc_argmin(flat, codebook, a2, b2)
    embed = _make_sc_gather(flat.shape[0], d)(
        codebook, idx_flat)
    return embed.reshape(*batch_shape, d), idx_flat.reshape(batch_shape)
